# baseline (speedup 1.0000x reference)
import functools

import jax
import jax.numpy as jnp
from jax import lax
from jax.experimental import pallas as pl
from jax.experimental.pallas import tpu as pltpu
from jax.experimental.pallas import tpu_sc as plsc

GLOBAL_MEAN = 3.5
D = 32
NC = 2
NS = 16
NW = NC * NS
L = 16
ICH = 128


def _mf_body(sid_hbm, pid_hbm, p_hbm, q_hbm, sb_hbm, pb_hbm, out_hbm,
             sid_v, pid_v, p_rows, q_rows, bs_v, bp_v, out_v, sem):
    kch = sid_v.shape[0]
    b_per_w = kch * ICH
    wid = lax.axis_index("s") * NC + lax.axis_index("c")
    base = wid * b_per_w

    pltpu.sync_copy(sid_hbm.at[wid], sid_v)
    pltpu.sync_copy(pid_hbm.at[wid], pid_v)

    copies = []
    for k in range(kch):
        idx_s = sid_v.at[k]
        idx_p = pid_v.at[k]
        sl = pl.ds(k * ICH, ICH)
        copies.append(pltpu.async_copy(p_hbm.at[idx_s], p_rows.at[sl], sem))
        copies.append(pltpu.async_copy(q_hbm.at[idx_p], q_rows.at[sl], sem))
        copies.append(pltpu.async_copy(sb_hbm.at[idx_s], bs_v.at[sl], sem))
        copies.append(pltpu.async_copy(pb_hbm.at[idx_p], bp_v.at[sl], sem))
    for c in copies:
        c.wait()

    lane = lax.iota(jnp.int32, L)

    def group(g, carry):
        e0 = g * L
        rows = e0 + lane
        acc = bs_v[pl.ds(e0, L)] + bp_v[pl.ds(e0, L)] + GLOBAL_MEAN
        for d in range(D):
            col = jnp.full((L,), d, jnp.int32)
            acc = acc + (plsc.load_gather(p_rows, [rows, col])
                         * plsc.load_gather(q_rows, [rows, col]))
        out_v[pl.ds(e0, L)] = acc
        return carry

    lax.fori_loop(0, b_per_w // L, group, 0)
    pltpu.sync_copy(out_v, out_hbm.at[pl.ds(base, b_per_w)])


@jax.jit
def kernel(SIDs, PIDs, P, Q, scientist_bias, paper_bias):
    B = SIDs.shape[0]
    b_per_w = B // NW
    kch = b_per_w // ICH
    sids = SIDs.astype(jnp.int32).reshape(NW, kch, ICH)
    pids = PIDs.astype(jnp.int32).reshape(NW, kch, ICH)
    sb = scientist_bias.reshape(-1)
    pb = paper_bias.reshape(-1)

    mesh = plsc.VectorSubcoreMesh(core_axis_name="c", subcore_axis_name="s")
    f = pl.kernel(
        _mf_body,
        out_type=jax.ShapeDtypeStruct((B,), jnp.float32),
        mesh=mesh,
        compiler_params=pltpu.CompilerParams(
            needs_layout_passes=False, use_tc_tiling_on_sc=False),
        scratch_types=[
            pltpu.VMEM((kch, ICH), jnp.int32),
            pltpu.VMEM((kch, ICH), jnp.int32),
            pltpu.VMEM((b_per_w, D), jnp.float32),
            pltpu.VMEM((b_per_w, D), jnp.float32),
            pltpu.VMEM((b_per_w,), jnp.float32),
            pltpu.VMEM((b_per_w,), jnp.float32),
            pltpu.VMEM((b_per_w,), jnp.float32),
            pltpu.SemaphoreType.DMA,
        ],
    )
    return f(sids, pids, P, Q, sb, pb)
